# baseline (speedup 1.0000x reference)
import jax
import jax.numpy as jnp
from jax.experimental import pallas as pl
from jax.experimental.pallas import tpu as pltpu

_TT = 512


def _moe_body(x_ref, wg_ref, bg_ref, wef_ref, bep_ref,
              out_ref, scores_ref, xs_ref, s2_ref):
    D = x_ref.shape[1]
    E = wg_ref.shape[1]
    i = pl.program_id(0)
    p = jax.lax.rem(i, 2)
    q = 1 - p

    x = x_ref[...]
    logits = jnp.dot(x, wg_ref[...], preferred_element_type=jnp.float32)
    logits = logits + bg_ref[...]
    m = jnp.max(logits, axis=-1, keepdims=True)
    e = jnp.exp(logits - m)
    s = e / jnp.sum(e, axis=-1, keepdims=True)
    scores_ref[...] = s

    col = jax.lax.broadcasted_iota(jnp.int32, s.shape, 1)
    for k in range(E):
        s_k = jnp.sum(jnp.where(col == k, s, 0.0), axis=1, keepdims=True)
        xs_ref[p, :, k * D:(k + 1) * D] = (x * s_k).astype(jnp.bfloat16)
    s2_ref[p] = jnp.concatenate([s] * (128 // E), axis=1).astype(jnp.bfloat16)

    out_ref[...] = (
        jnp.dot(xs_ref[q], wef_ref[...], preferred_element_type=jnp.float32)
        + jnp.dot(s2_ref[q], bep_ref[...], preferred_element_type=jnp.float32)
    )


def kernel(x, Wg, bg, We, be):
    T, D = x.shape
    E = Wg.shape[1]
    n = T // _TT
    wef = We.reshape(E * D, D).astype(jnp.bfloat16)
    bep = jnp.zeros((128, D), jnp.bfloat16).at[:E].set(be.astype(jnp.bfloat16))

    out, scores = pl.pallas_call(
        _moe_body,
        grid=(n + 1,),
        in_specs=[
            pl.BlockSpec((_TT, D), lambda i: (jnp.minimum(i, n - 1), 0)),
            pl.BlockSpec((D, E), lambda i: (0, 0)),
            pl.BlockSpec((1, E), lambda i: (0, 0)),
            pl.BlockSpec((E * D, D), lambda i: (0, 0)),
            pl.BlockSpec((128, D), lambda i: (0, 0)),
        ],
        out_specs=[
            pl.BlockSpec((_TT, D), lambda i: (jnp.maximum(i - 1, 0), 0)),
            pl.BlockSpec((_TT, E), lambda i: (jnp.minimum(i, n - 1), 0)),
        ],
        out_shape=[
            jax.ShapeDtypeStruct((T, D), jnp.float32),
            jax.ShapeDtypeStruct((T, E), jnp.float32),
        ],
        scratch_shapes=[
            pltpu.VMEM((2, _TT, E * D), jnp.bfloat16),
            pltpu.VMEM((2, _TT, 128), jnp.bfloat16),
        ],
        compiler_params=pltpu.CompilerParams(
            dimension_semantics=("arbitrary",),
        ),
    )(x, Wg, bg.reshape(1, E), wef, bep)
    return out, scores


# speedup vs baseline: 1.1705x; 1.1705x over previous
import jax
import jax.numpy as jnp
from jax.experimental import pallas as pl
from jax.experimental.pallas import tpu as pltpu

_TT = 1024


def _moe_body(x_ref, wg_ref, bg_ref, wef_ref, bep_ref,
              out_ref, scores_ref, xs_ref):
    D = x_ref.shape[1]
    E = wg_ref.shape[1]

    x = x_ref[...]
    logits = jnp.dot(x, wg_ref[...], preferred_element_type=jnp.float32)
    logits = logits + bg_ref[...]
    m = jnp.max(logits, axis=-1, keepdims=True)
    e = jnp.exp(logits - m)
    s = e / jnp.sum(e, axis=-1, keepdims=True)
    scores_ref[...] = s

    col = jax.lax.broadcasted_iota(jnp.int32, s.shape, 1)
    for k in range(E):
        s_k = jnp.sum(jnp.where(col == k, s, 0.0), axis=1, keepdims=True)
        xs_ref[:, k * D:(k + 1) * D] = (x * s_k).astype(jnp.bfloat16)

    s128 = jnp.concatenate([s] * (128 // E), axis=1).astype(jnp.bfloat16)
    out_ref[...] = (
        jnp.dot(xs_ref[...], wef_ref[...], preferred_element_type=jnp.float32)
        + jnp.dot(s128, bep_ref[...], preferred_element_type=jnp.float32)
    )


def kernel(x, Wg, bg, We, be):
    T, D = x.shape
    E = Wg.shape[1]
    wef = We.reshape(E * D, D).astype(jnp.bfloat16)
    bep = jnp.zeros((128, D), jnp.bfloat16).at[:E].set(be.astype(jnp.bfloat16))

    out, scores = pl.pallas_call(
        _moe_body,
        grid=(T // _TT,),
        in_specs=[
            pl.BlockSpec((_TT, D), lambda i: (i, 0)),
            pl.BlockSpec((D, E), lambda i: (0, 0)),
            pl.BlockSpec((1, E), lambda i: (0, 0)),
            pl.BlockSpec((E * D, D), lambda i: (0, 0)),
            pl.BlockSpec((128, D), lambda i: (0, 0)),
        ],
        out_specs=[
            pl.BlockSpec((_TT, D), lambda i: (i, 0)),
            pl.BlockSpec((_TT, E), lambda i: (i, 0)),
        ],
        out_shape=[
            jax.ShapeDtypeStruct((T, D), jnp.float32),
            jax.ShapeDtypeStruct((T, E), jnp.float32),
        ],
        scratch_shapes=[pltpu.VMEM((_TT, E * D), jnp.bfloat16)],
        compiler_params=pltpu.CompilerParams(
            dimension_semantics=("parallel",),
        ),
    )(x, Wg, bg.reshape(1, E), wef, bep)
    return out, scores


# speedup vs baseline: 1.2507x; 1.0686x over previous
import jax
import jax.numpy as jnp
from jax.experimental import pallas as pl
from jax.experimental.pallas import tpu as pltpu

_TT = 512


def _moe_body(x_ref, wg_ref, bg_ref, we_ref, be_ref,
              out_ref, scores_ref, xs_ref, wef_ref, bep_ref, s2_ref):
    D = x_ref.shape[1]
    E = wg_ref.shape[1]
    i = pl.program_id(0)
    k = pl.program_id(1)

    @pl.when(i == 0)
    def _cast_chunk():
        wef_ref[pl.ds(k * D, D), :] = we_ref[0].astype(jnp.bfloat16)

    @pl.when((i == 0) & (k == 0))
    def _bias_pad():
        bep_ref[...] = jnp.concatenate(
            [be_ref[...].astype(jnp.bfloat16),
             jnp.zeros((128 - E, D), jnp.bfloat16)], axis=0)

    @pl.when((i > 0) & (k == 0))
    def _gate_and_build():
        x = x_ref[...]
        logits = jnp.dot(x, wg_ref[...], preferred_element_type=jnp.float32)
        logits = logits + bg_ref[...]
        m = jnp.max(logits, axis=-1, keepdims=True)
        e = jnp.exp(logits - m)
        s = e / jnp.sum(e, axis=-1, keepdims=True)
        scores_ref[...] = s
        col = jax.lax.broadcasted_iota(jnp.int32, s.shape, 1)
        for kk in range(E):
            s_kk = jnp.sum(jnp.where(col == kk, s, 0.0), axis=1,
                           keepdims=True)
            xs_ref[:, kk * D:(kk + 1) * D] = (x * s_kk).astype(jnp.bfloat16)
        s2_ref[...] = jnp.concatenate([s] * (128 // E),
                                      axis=1).astype(jnp.bfloat16)

    @pl.when((i > 0) & (k == E - 1))
    def _dot():
        out_ref[...] = (
            jnp.dot(xs_ref[...], wef_ref[...],
                    preferred_element_type=jnp.float32)
            + jnp.dot(s2_ref[...], bep_ref[...],
                      preferred_element_type=jnp.float32)
        )


def kernel(x, Wg, bg, We, be):
    T, D = x.shape
    E = Wg.shape[1]
    n = T // _TT

    out, scores = pl.pallas_call(
        _moe_body,
        grid=(n + 1, E),
        in_specs=[
            pl.BlockSpec((_TT, D), lambda i, k: (jnp.maximum(i - 1, 0), 0)),
            pl.BlockSpec((D, E), lambda i, k: (0, 0)),
            pl.BlockSpec((1, E), lambda i, k: (0, 0)),
            pl.BlockSpec((1, D, D),
                         lambda i, k: (jnp.where(i == 0, k, E - 1), 0, 0)),
            pl.BlockSpec((E, D), lambda i, k: (0, 0)),
        ],
        out_specs=[
            pl.BlockSpec((_TT, D), lambda i, k: (jnp.maximum(i - 1, 0), 0)),
            pl.BlockSpec((_TT, E), lambda i, k: (jnp.maximum(i - 1, 0), 0)),
        ],
        out_shape=[
            jax.ShapeDtypeStruct((T, D), jnp.float32),
            jax.ShapeDtypeStruct((T, E), jnp.float32),
        ],
        scratch_shapes=[
            pltpu.VMEM((_TT, E * D), jnp.bfloat16),
            pltpu.VMEM((E * D, D), jnp.bfloat16),
            pltpu.VMEM((128, D), jnp.bfloat16),
            pltpu.VMEM((_TT, 128), jnp.bfloat16),
        ],
        compiler_params=pltpu.CompilerParams(
            dimension_semantics=("arbitrary", "arbitrary"),
        ),
    )(x, Wg, bg.reshape(1, E), We, be)
    return out, scores


# speedup vs baseline: 1.3757x; 1.0999x over previous
import jax
import jax.numpy as jnp
from jax.experimental import pallas as pl
from jax.experimental.pallas import tpu as pltpu

_TT = 512


def _moe_body(x_ref, wg_ref, bg_ref, we_ref, be_ref,
              out_ref, scores_ref, xs_ref, wef_ref, bep_ref, s2_ref):
    D = x_ref.shape[1]
    E = wg_ref.shape[1]
    TT = x_ref.shape[0]
    i = pl.program_id(0)

    @pl.when(i < E)
    def _cast_chunk():
        wef_ref[pl.ds(i * D, D), :] = we_ref[0].astype(jnp.bfloat16)

    @pl.when(i == 0)
    def _bias_pad():
        bep_ref[...] = jnp.concatenate(
            [be_ref[...].astype(jnp.bfloat16),
             jnp.zeros((128 - E, D), jnp.bfloat16)], axis=0)

    @pl.when(i >= E)
    def _tile():
        x = x_ref[...]
        logits = jnp.dot(x, wg_ref[...], preferred_element_type=jnp.float32)
        logits = logits + bg_ref[...]
        m = jnp.max(logits, axis=-1, keepdims=True)
        e = jnp.exp(logits - m)
        s = e / jnp.sum(e, axis=-1, keepdims=True)
        scores_ref[...] = s
        s2_ref[...] = jnp.concatenate([s] * (128 // E),
                                      axis=1).astype(jnp.bfloat16)
        col = jax.lax.broadcasted_iota(jnp.int32, (TT // 2, E), 1)
        for h in range(2):
            r = pl.ds(h * (TT // 2), TT // 2)
            sh = s[h * (TT // 2):(h + 1) * (TT // 2)]
            xh = x[h * (TT // 2):(h + 1) * (TT // 2)]
            for kk in range(E):
                s_kk = jnp.sum(jnp.where(col == kk, sh, 0.0), axis=1,
                               keepdims=True)
                xs_ref[r, kk * D:(kk + 1) * D] = (xh * s_kk).astype(
                    jnp.bfloat16)
            out_ref[r, :] = (
                jnp.dot(xs_ref[r, :], wef_ref[...],
                        preferred_element_type=jnp.float32)
                + jnp.dot(s2_ref[r, :], bep_ref[...],
                          preferred_element_type=jnp.float32)
            )


def kernel(x, Wg, bg, We, be):
    T, D = x.shape
    E = Wg.shape[1]
    n = T // _TT

    out, scores = pl.pallas_call(
        _moe_body,
        grid=(E + n,),
        in_specs=[
            pl.BlockSpec((_TT, D), lambda i: (jnp.maximum(i - E, 0), 0)),
            pl.BlockSpec((D, E), lambda i: (0, 0)),
            pl.BlockSpec((1, E), lambda i: (0, 0)),
            pl.BlockSpec((1, D, D),
                         lambda i: (jnp.minimum(i, E - 1), 0, 0)),
            pl.BlockSpec((E, D), lambda i: (0, 0)),
        ],
        out_specs=[
            pl.BlockSpec((_TT, D), lambda i: (jnp.maximum(i - E, 0), 0)),
            pl.BlockSpec((_TT, E), lambda i: (jnp.maximum(i - E, 0), 0)),
        ],
        out_shape=[
            jax.ShapeDtypeStruct((T, D), jnp.float32),
            jax.ShapeDtypeStruct((T, E), jnp.float32),
        ],
        scratch_shapes=[
            pltpu.VMEM((_TT, E * D), jnp.bfloat16),
            pltpu.VMEM((E * D, D), jnp.bfloat16),
            pltpu.VMEM((128, D), jnp.bfloat16),
            pltpu.VMEM((_TT, 128), jnp.bfloat16),
        ],
        compiler_params=pltpu.CompilerParams(
            dimension_semantics=("arbitrary",),
        ),
    )(x, Wg, bg.reshape(1, E), We, be)
    return out, scores
